# baseline (speedup 1.0000x reference)
import functools

import jax
import jax.numpy as jnp
from jax import lax
from jax.experimental import pallas as pl
from jax.experimental.pallas import tpu as pltpu
from jax.experimental.pallas import tpu_sc as plsc

NC = 2
NS = 16
LANES = 16
K = 128
TCB = 400


def _vector_mesh():
    return plsc.VectorSubcoreMesh(core_axis_name="c", subcore_axis_name="s")


def _fill(ref, rows, width, value):
    @pl.loop(0, rows)
    def _(i):
        @pl.loop(0, width, step=LANES)
        def _(j):
            ref.at[i].at[pl.ds(j, LANES)][...] = jnp.full((LANES,), value,
                                                          jnp.float32)


def _deg_call(dst2d, npad, width):
    n_rows = dst2d.shape[0]
    n_chunks = n_rows // (NC * NS)
    stripe = npad // NS

    @functools.partial(
        pl.kernel,
        out_type=jax.ShapeDtypeStruct((NC, npad, width), jnp.float32),
        mesh=_vector_mesh(),
        scratch_types=[
            pltpu.VMEM((n_chunks, K), jnp.int32),
            pltpu.VMEM((K, width), jnp.float32),
            pltpu.VMEM_SHARED((npad, width), jnp.float32),
            pltpu.SemaphoreType.DMA,
            pltpu.SemaphoreType.DMA,
            pltpu.SemaphoreType.DMA,
            pltpu.SemaphoreType.DMA,
        ],
    )
    def k(dst_hbm, out_hbm, idx_v, ones_v, cnt_sh, s0, s1, s2, s3):
        sems = (s0, s1, s2, s3)
        cid = lax.axis_index("c")
        sid = lax.axis_index("s")
        row_base = (cid * NS + sid) * n_chunks
        pltpu.sync_copy(dst_hbm.at[pl.ds(row_base, n_chunks)], idx_v)

        _fill(ones_v, K, width, 0.0)
        n_full = stripe // K
        tail = stripe - n_full * K

        @pl.loop(0, n_full)
        def _(t):
            pltpu.sync_copy(ones_v, cnt_sh.at[pl.ds(sid * stripe + t * K, K)])
        if tail:
            pltpu.sync_copy(ones_v.at[pl.ds(0, tail)],
                            cnt_sh.at[pl.ds(sid * stripe + n_full * K, tail)])

        _fill(ones_v, K, width, 1.0)

        plsc.subcore_barrier()

        @pl.loop(0, n_chunks, step=4)
        def _(ci):
            for j in range(4):
                @pl.when(ci + j - 4 >= 0)
                def _():
                    pltpu.make_async_copy(
                        ones_v, cnt_sh.at[idx_v.at[ci + j - 4]],
                        sems[j]).wait()
                pltpu.async_copy(ones_v, cnt_sh.at[idx_v.at[ci + j]],
                                 sems[j], add=True)
        for j in range(4):
            pltpu.make_async_copy(
                ones_v, cnt_sh.at[idx_v.at[n_chunks - 4 + j]], sems[j]).wait()

        plsc.subcore_barrier()

        @pl.loop(0, n_full)
        def _(t):
            r = sid * stripe + t * K
            pltpu.sync_copy(cnt_sh.at[pl.ds(r, K)],
                            out_hbm.at[cid].at[pl.ds(r, K)])
        if tail:
            r = sid * stripe + n_full * K
            pltpu.sync_copy(cnt_sh.at[pl.ds(r, tail)],
                            out_hbm.at[cid].at[pl.ds(r, tail)])

    return k(dst2d)


def _agg_call(p, ei2, npad, width):
    n_rows = ei2.shape[0]
    n_chunks = n_rows // (NC * NS)
    assert n_chunks % 2 == 0
    stripe = npad // NS

    @functools.partial(
        pl.kernel,
        out_type=jax.ShapeDtypeStruct((NC, npad, width), jnp.float32),
        mesh=_vector_mesh(),
        scratch_types=[
            pltpu.VMEM((2, K), jnp.int32),
            pltpu.VMEM((2, K), jnp.int32),
            pltpu.VMEM((K, width), jnp.float32),
            pltpu.VMEM((K, width), jnp.float32),
            pltpu.VMEM_SHARED((npad, width), jnp.float32),
            pltpu.SemaphoreType.DMA,
            pltpu.SemaphoreType.DMA,
            pltpu.SemaphoreType.DMA,
            pltpu.SemaphoreType.DMA,
            pltpu.SemaphoreType.DMA,
            pltpu.SemaphoreType.DMA,
        ],
    )
    def k(p_hbm, ei_hbm, out_hbm, i_a, i_b, rows_a, rows_b, s_sh,
          sem_ia, sem_ib, sem_a, sem_b, sem_sa, sem_sb):
        cid = lax.axis_index("c")
        sid = lax.axis_index("s")
        base = (cid * NS + sid) * n_chunks

        _fill(rows_a, K, width, 0.0)
        n_full = stripe // K
        tail = stripe - n_full * K

        @pl.loop(0, n_full)
        def _(t):
            pltpu.sync_copy(rows_a, s_sh.at[pl.ds(sid * stripe + t * K, K)])
        if tail:
            pltpu.sync_copy(rows_a.at[pl.ds(0, tail)],
                            s_sh.at[pl.ds(sid * stripe + n_full * K, tail)])

        plsc.subcore_barrier()

        pltpu.sync_copy(ei_hbm.at[base], i_a)
        pltpu.async_copy(ei_hbm.at[base + 1], i_b, sem_ib)
        pltpu.async_copy(p_hbm.at[i_a.at[0]], rows_a, sem_a)
        pltpu.make_async_copy(ei_hbm.at[base + 1], i_b, sem_ib).wait()
        pltpu.async_copy(p_hbm.at[i_b.at[0]], rows_b, sem_b)

        @pl.loop(0, n_chunks, step=2)
        def _(c):
            pltpu.make_async_copy(p_hbm.at[i_a.at[0]], rows_a, sem_a).wait()
            pltpu.async_copy(rows_a, s_sh.at[i_a.at[1]], sem_sa, add=True)
            pltpu.make_async_copy(p_hbm.at[i_b.at[0]], rows_b, sem_b).wait()
            pltpu.async_copy(rows_b, s_sh.at[i_b.at[1]], sem_sb, add=True)

            @pl.when(c + 2 < n_chunks)
            def _():
                pltpu.make_async_copy(rows_a, s_sh.at[i_a.at[1]],
                                      sem_sa).wait()
                pltpu.async_copy(ei_hbm.at[base + c + 2], i_a, sem_ia)
                pltpu.make_async_copy(ei_hbm.at[base + c + 2], i_a,
                                      sem_ia).wait()
                pltpu.async_copy(p_hbm.at[i_a.at[0]], rows_a, sem_a)

            @pl.when(c + 3 < n_chunks)
            def _():
                pltpu.make_async_copy(rows_b, s_sh.at[i_b.at[1]],
                                      sem_sb).wait()
                pltpu.async_copy(ei_hbm.at[base + c + 3], i_b, sem_ib)
                pltpu.make_async_copy(ei_hbm.at[base + c + 3], i_b,
                                      sem_ib).wait()
                pltpu.async_copy(p_hbm.at[i_b.at[0]], rows_b, sem_b)

        pltpu.make_async_copy(rows_a, s_sh.at[i_a.at[1]], sem_sa).wait()
        pltpu.make_async_copy(rows_b, s_sh.at[i_b.at[1]], sem_sb).wait()

        plsc.subcore_barrier()

        @pl.loop(0, n_full)
        def _(t):
            r = sid * stripe + t * K
            pltpu.sync_copy(s_sh.at[pl.ds(r, K)],
                            out_hbm.at[cid].at[pl.ds(r, K)])
        if tail:
            r = sid * stripe + n_full * K
            pltpu.sync_copy(s_sh.at[pl.ds(r, tail)],
                            out_hbm.at[cid].at[pl.ds(r, tail)])

    return k(p, ei2)


def _dinv_block(c0, c1):
    deg = c0[0, :, 0] + c1[0, :, 0] + 1.0
    return lax.rsqrt(deg)


def _p0_call(x, w0, cnt, npad):
    n, d = x.shape
    h = w0.shape[1]

    def body(x_ref, w_ref, c0_ref, c1_ref, p_ref):
        dinv = _dinv_block(c0_ref, c1_ref)
        hw = jnp.dot(x_ref[...], w_ref[...], preferred_element_type=jnp.float32)
        p_ref[...] = hw * dinv[:, None]

    return pl.pallas_call(
        body,
        grid=(n // TCB,),
        in_specs=[
            pl.BlockSpec((TCB, d), lambda i: (i, 0)),
            pl.BlockSpec((d, h), lambda i: (0, 0)),
            pl.BlockSpec((1, TCB, 128), lambda i: (0, i, 0)),
            pl.BlockSpec((1, TCB, 128), lambda i: (1, i, 0)),
        ],
        out_specs=pl.BlockSpec((TCB, h), lambda i: (i, 0)),
        out_shape=jax.ShapeDtypeStruct((n, h), jnp.float32),
    )(x, w0, cnt, cnt)


def _p1_call(s0, p0, cnt, b0, npad):
    n, h = p0.shape

    def body(s0a, s0b, p0_ref, c0_ref, c1_ref, b_ref, p1_ref):
        dinv = _dinv_block(c0_ref, c1_ref)
        hmat = (s0a[0] + s0b[0] + p0_ref[...]) * dinv[:, None] + b_ref[...]
        hmat = jnp.maximum(hmat, 0.0)
        p1_ref[...] = hmat * dinv[:, None]

    return pl.pallas_call(
        body,
        grid=(n // TCB,),
        in_specs=[
            pl.BlockSpec((1, TCB, h), lambda i: (0, i, 0)),
            pl.BlockSpec((1, TCB, h), lambda i: (1, i, 0)),
            pl.BlockSpec((TCB, h), lambda i: (i, 0)),
            pl.BlockSpec((1, TCB, 128), lambda i: (0, i, 0)),
            pl.BlockSpec((1, TCB, 128), lambda i: (1, i, 0)),
            pl.BlockSpec((1, h), lambda i: (0, 0)),
        ],
        out_specs=pl.BlockSpec((TCB, h), lambda i: (i, 0)),
        out_shape=jax.ShapeDtypeStruct((n, h), jnp.float32),
    )(s0, s0, p0, cnt, cnt, b0)


def _out_call(s1, p1, cnt, w1, b1, npad):
    n, h = p1.shape
    c = w1.shape[1]

    def body(s1a, s1b, p1_ref, c0_ref, c1_ref, w_ref, b_ref, o_ref):
        dinv = _dinv_block(c0_ref, c1_ref)
        agg = (s1a[0] + s1b[0] + p1_ref[...]) * dinv[:, None]
        o_ref[...] = jnp.dot(agg, w_ref[...],
                             preferred_element_type=jnp.float32) + b_ref[...]

    return pl.pallas_call(
        body,
        grid=(n // TCB,),
        in_specs=[
            pl.BlockSpec((1, TCB, h), lambda i: (0, i, 0)),
            pl.BlockSpec((1, TCB, h), lambda i: (1, i, 0)),
            pl.BlockSpec((TCB, h), lambda i: (i, 0)),
            pl.BlockSpec((1, TCB, 128), lambda i: (0, i, 0)),
            pl.BlockSpec((1, TCB, 128), lambda i: (1, i, 0)),
            pl.BlockSpec((h, c), lambda i: (0, 0)),
            pl.BlockSpec((1, c), lambda i: (0, 0)),
        ],
        out_specs=pl.BlockSpec((TCB, c), lambda i: (i, 0)),
        out_shape=jax.ShapeDtypeStruct((n, c), jnp.float32),
    )(s1, s1, p1, cnt, cnt, w1, b1)


def kernel(x, edge_index, W0, b0, W1, b1):
    n, d = x.shape
    h = W0.shape[1]

    src, dst = edge_index[0], edge_index[1]
    e = src.shape[0]
    chunk_total = NC * NS * K * 2
    epad = ((e + chunk_total - 1) // chunk_total) * chunk_total
    npad = ((n + 1 + 127) // 128) * 128

    pad = epad - e
    arp = jnp.arange(pad, dtype=src.dtype)
    src_p = jnp.concatenate([src, n + arp % K]).reshape(-1, K)
    dst_p = jnp.concatenate([dst, arp % n]).reshape(-1, K)
    dst_deg = jnp.concatenate([dst, n + arp % (npad - n)]).reshape(-1, K)
    ei2 = jnp.stack([src_p, dst_p], axis=1)
    b0r = b0.reshape(1, h)
    b1r = b1.reshape(1, b1.shape[0])
    zrows = jnp.zeros((K, h), jnp.float32)

    cnt = _deg_call(dst_deg, npad, h)
    p0 = _p0_call(x, W0, cnt, npad)
    s0 = _agg_call(jnp.concatenate([p0, zrows]), ei2, npad, h)
    p1 = _p1_call(s0, p0, cnt, b0r, npad)
    s1 = _agg_call(jnp.concatenate([p1, zrows]), ei2, npad, h)
    return _out_call(s1, p1, cnt, W1, b1r, npad)


# speedup vs baseline: 1.0448x; 1.0448x over previous
import functools

import jax
import jax.numpy as jnp
from jax import lax
from jax.experimental import pallas as pl
from jax.experimental.pallas import tpu as pltpu
from jax.experimental.pallas import tpu_sc as plsc

NC = 2
NS = 16
LANES = 16
K = 64
TCB = 400


def _vector_mesh():
    return plsc.VectorSubcoreMesh(core_axis_name="c", subcore_axis_name="s")


def _fill(ref, rows, width, value):
    @pl.loop(0, rows)
    def _(i):
        @pl.loop(0, width, step=LANES)
        def _(j):
            ref.at[i].at[pl.ds(j, LANES)][...] = jnp.full((LANES,), value,
                                                          jnp.float32)


def _deg_call(dst2d, npad, width):
    n_rows = dst2d.shape[0]
    n_chunks = n_rows // (NC * NS)
    stripe = npad // NS

    @functools.partial(
        pl.kernel,
        out_type=jax.ShapeDtypeStruct((NC, npad, width), jnp.float32),
        mesh=_vector_mesh(),
        scratch_types=[
            pltpu.VMEM((n_chunks, K), jnp.int32),
            pltpu.VMEM((K, width), jnp.float32),
            pltpu.VMEM_SHARED((npad, width), jnp.float32),
            pltpu.SemaphoreType.DMA,
            pltpu.SemaphoreType.DMA,
            pltpu.SemaphoreType.DMA,
            pltpu.SemaphoreType.DMA,
        ],
    )
    def k(dst_hbm, out_hbm, idx_v, ones_v, cnt_sh, s0, s1, s2, s3):
        sems = (s0, s1, s2, s3)
        cid = lax.axis_index("c")
        sid = lax.axis_index("s")
        row_base = (cid * NS + sid) * n_chunks
        pltpu.sync_copy(dst_hbm.at[pl.ds(row_base, n_chunks)], idx_v)

        _fill(ones_v, K, width, 0.0)
        n_full = stripe // K
        tail = stripe - n_full * K

        @pl.loop(0, n_full)
        def _(t):
            pltpu.sync_copy(ones_v, cnt_sh.at[pl.ds(sid * stripe + t * K, K)])
        if tail:
            pltpu.sync_copy(ones_v.at[pl.ds(0, tail)],
                            cnt_sh.at[pl.ds(sid * stripe + n_full * K, tail)])

        _fill(ones_v, K, width, 1.0)

        plsc.subcore_barrier()

        @pl.loop(0, n_chunks, step=4)
        def _(ci):
            for j in range(4):
                @pl.when(ci + j - 4 >= 0)
                def _():
                    pltpu.make_async_copy(
                        ones_v, cnt_sh.at[idx_v.at[ci + j - 4]],
                        sems[j]).wait()
                pltpu.async_copy(ones_v, cnt_sh.at[idx_v.at[ci + j]],
                                 sems[j], add=True)
        for j in range(4):
            pltpu.make_async_copy(
                ones_v, cnt_sh.at[idx_v.at[n_chunks - 4 + j]], sems[j]).wait()

        plsc.subcore_barrier()

        @pl.loop(0, n_full)
        def _(t):
            r = sid * stripe + t * K
            pltpu.sync_copy(cnt_sh.at[pl.ds(r, K)],
                            out_hbm.at[cid].at[pl.ds(r, K)])
        if tail:
            r = sid * stripe + n_full * K
            pltpu.sync_copy(cnt_sh.at[pl.ds(r, tail)],
                            out_hbm.at[cid].at[pl.ds(r, tail)])

    return k(dst2d)


def _agg_call(p, ei2, npad, width):
    n_rows = ei2.shape[0]
    n_chunks = n_rows // (NC * NS)
    assert n_chunks % 8 == 0
    stripe = npad // NS

    @functools.partial(
        pl.kernel,
        out_type=jax.ShapeDtypeStruct((NC, npad, width), jnp.float32),
        mesh=_vector_mesh(),
        scratch_types=(
            [pltpu.VMEM((2, K), jnp.int32)] * 8
            + [pltpu.VMEM((K, width), jnp.float32)] * 4
            + [pltpu.VMEM_SHARED((npad, width), jnp.float32)]
            + [pltpu.SemaphoreType.DMA] * 16
        ),
    )
    def k(p_hbm, ei_hbm, out_hbm, *refs):
        idx = refs[0:8]
        rows = refs[8:12]
        s_sh = refs[12]
        g = refs[13:17]
        s = refs[17:21]
        si = refs[21:29]
        cid = lax.axis_index("c")
        sid = lax.axis_index("s")
        base = (cid * NS + sid) * n_chunks

        _fill(rows[0], K, width, 0.0)
        n_full = stripe // K
        tail = stripe - n_full * K

        @pl.loop(0, n_full)
        def _(t):
            pltpu.sync_copy(rows[0], s_sh.at[pl.ds(sid * stripe + t * K, K)])
        if tail:
            pltpu.sync_copy(rows[0].at[pl.ds(0, tail)],
                            s_sh.at[pl.ds(sid * stripe + n_full * K, tail)])

        plsc.subcore_barrier()

        for q in range(6):
            pltpu.async_copy(ei_hbm.at[base + q], idx[q], si[q])
        for b in range(2):
            pltpu.make_async_copy(ei_hbm.at[base + b], idx[b], si[b]).wait()
            pltpu.async_copy(p_hbm.at[idx[b].at[0]], rows[b], g[b])

        @pl.loop(0, n_chunks, step=8)
        def _(c):
            for u in range(8):
                b = u % 4
                br = (u + 2) % 4
                qr = (u + 2) % 8
                qp = (u + 6) % 8

                pltpu.make_async_copy(p_hbm.at[idx[u].at[0]], rows[b],
                                      g[b]).wait()
                pltpu.async_copy(rows[b], s_sh.at[idx[u].at[1]], s[b],
                                 add=True)

                @pl.when(c + u + 2 < n_chunks)
                def _():
                    @pl.when(c + u - 2 >= 0)
                    def _():
                        pltpu.make_async_copy(rows[br],
                                              s_sh.at[idx[qr].at[1]],
                                              s[br]).wait()
                    pltpu.make_async_copy(ei_hbm.at[base + c + u + 2],
                                          idx[qr], si[qr]).wait()
                    pltpu.async_copy(p_hbm.at[idx[qr].at[0]], rows[br], g[br])

                @pl.when(c + u + 6 < n_chunks)
                def _():
                    pltpu.async_copy(ei_hbm.at[base + c + u + 6], idx[qp],
                                     si[qp])

        for b in range(4):
            pltpu.make_async_copy(rows[b], s_sh.at[idx[4 + b].at[1]],
                                  s[b]).wait()

        plsc.subcore_barrier()

        @pl.loop(0, n_full)
        def _(t):
            r = sid * stripe + t * K
            pltpu.sync_copy(s_sh.at[pl.ds(r, K)],
                            out_hbm.at[cid].at[pl.ds(r, K)])
        if tail:
            r = sid * stripe + n_full * K
            pltpu.sync_copy(s_sh.at[pl.ds(r, tail)],
                            out_hbm.at[cid].at[pl.ds(r, tail)])

    return k(p, ei2)


def _dinv_block(c0, c1):
    deg = c0[0, :, 0] + c1[0, :, 0] + 1.0
    return lax.rsqrt(deg)


def _p0_call(x, w0, cnt, npad):
    n, d = x.shape
    h = w0.shape[1]

    def body(x_ref, w_ref, c0_ref, c1_ref, p_ref):
        dinv = _dinv_block(c0_ref, c1_ref)
        hw = jnp.dot(x_ref[...], w_ref[...], preferred_element_type=jnp.float32)
        p_ref[...] = hw * dinv[:, None]

    return pl.pallas_call(
        body,
        grid=(n // TCB,),
        in_specs=[
            pl.BlockSpec((TCB, d), lambda i: (i, 0)),
            pl.BlockSpec((d, h), lambda i: (0, 0)),
            pl.BlockSpec((1, TCB, 128), lambda i: (0, i, 0)),
            pl.BlockSpec((1, TCB, 128), lambda i: (1, i, 0)),
        ],
        out_specs=pl.BlockSpec((TCB, h), lambda i: (i, 0)),
        out_shape=jax.ShapeDtypeStruct((n, h), jnp.float32),
    )(x, w0, cnt, cnt)


def _p1_call(s0, p0, cnt, b0, npad):
    n, h = p0.shape

    def body(s0a, s0b, p0_ref, c0_ref, c1_ref, b_ref, p1_ref):
        dinv = _dinv_block(c0_ref, c1_ref)
        hmat = (s0a[0] + s0b[0] + p0_ref[...]) * dinv[:, None] + b_ref[...]
        hmat = jnp.maximum(hmat, 0.0)
        p1_ref[...] = hmat * dinv[:, None]

    return pl.pallas_call(
        body,
        grid=(n // TCB,),
        in_specs=[
            pl.BlockSpec((1, TCB, h), lambda i: (0, i, 0)),
            pl.BlockSpec((1, TCB, h), lambda i: (1, i, 0)),
            pl.BlockSpec((TCB, h), lambda i: (i, 0)),
            pl.BlockSpec((1, TCB, 128), lambda i: (0, i, 0)),
            pl.BlockSpec((1, TCB, 128), lambda i: (1, i, 0)),
            pl.BlockSpec((1, h), lambda i: (0, 0)),
        ],
        out_specs=pl.BlockSpec((TCB, h), lambda i: (i, 0)),
        out_shape=jax.ShapeDtypeStruct((n, h), jnp.float32),
    )(s0, s0, p0, cnt, cnt, b0)


def _out_call(s1, p1, cnt, w1, b1, npad):
    n, h = p1.shape
    c = w1.shape[1]

    def body(s1a, s1b, p1_ref, c0_ref, c1_ref, w_ref, b_ref, o_ref):
        dinv = _dinv_block(c0_ref, c1_ref)
        agg = (s1a[0] + s1b[0] + p1_ref[...]) * dinv[:, None]
        o_ref[...] = jnp.dot(agg, w_ref[...],
                             preferred_element_type=jnp.float32) + b_ref[...]

    return pl.pallas_call(
        body,
        grid=(n // TCB,),
        in_specs=[
            pl.BlockSpec((1, TCB, h), lambda i: (0, i, 0)),
            pl.BlockSpec((1, TCB, h), lambda i: (1, i, 0)),
            pl.BlockSpec((TCB, h), lambda i: (i, 0)),
            pl.BlockSpec((1, TCB, 128), lambda i: (0, i, 0)),
            pl.BlockSpec((1, TCB, 128), lambda i: (1, i, 0)),
            pl.BlockSpec((h, c), lambda i: (0, 0)),
            pl.BlockSpec((1, c), lambda i: (0, 0)),
        ],
        out_specs=pl.BlockSpec((TCB, c), lambda i: (i, 0)),
        out_shape=jax.ShapeDtypeStruct((n, c), jnp.float32),
    )(s1, s1, p1, cnt, cnt, w1, b1)


def kernel(x, edge_index, W0, b0, W1, b1):
    n, d = x.shape
    h = W0.shape[1]

    src, dst = edge_index[0], edge_index[1]
    e = src.shape[0]
    chunk_total = NC * NS * K * 8
    epad = ((e + chunk_total - 1) // chunk_total) * chunk_total
    npad = ((n + 1 + 127) // 128) * 128

    pad = epad - e
    arp = jnp.arange(pad, dtype=src.dtype)
    src_p = jnp.concatenate([src, n + arp % K]).reshape(-1, K)
    dst_p = jnp.concatenate([dst, arp % n]).reshape(-1, K)
    dst_deg = jnp.concatenate([dst, n + arp % (npad - n)]).reshape(-1, K)
    ei2 = jnp.stack([src_p, dst_p], axis=1)
    b0r = b0.reshape(1, h)
    b1r = b1.reshape(1, b1.shape[0])
    zrows = jnp.zeros((K, h), jnp.float32)

    cnt = _deg_call(dst_deg, npad, h)
    p0 = _p0_call(x, W0, cnt, npad)
    s0 = _agg_call(jnp.concatenate([p0, zrows]), ei2, npad, h)
    p1 = _p1_call(s0, p0, cnt, b0r, npad)
    s1 = _agg_call(jnp.concatenate([p1, zrows]), ei2, npad, h)
    return _out_call(s1, p1, cnt, W1, b1r, npad)


# speedup vs baseline: 1.1250x; 1.0768x over previous
import dataclasses
import functools

import jax
import jax.numpy as jnp
from jax import lax
from jax.experimental import pallas as pl
from jax.experimental.pallas import tpu as pltpu
from jax.experimental.pallas import tpu_sc as plsc

NC = 2
NS = 16
LANES = 16
K = 64
TCB = 400


def _vector_mesh():
    return plsc.VectorSubcoreMesh(core_axis_name="c", subcore_axis_name="s")


def _fill(ref, rows, width, value):
    @pl.loop(0, rows)
    def _(i):
        @pl.loop(0, width, step=LANES)
        def _(j):
            ref.at[i].at[pl.ds(j, LANES)][...] = jnp.full((LANES,), value,
                                                          jnp.float32)


def _deg_call(dst2d, npad, width):
    n_rows = dst2d.shape[0]
    n_chunks = n_rows // (NC * NS)
    stripe = npad // NS

    @functools.partial(
        pl.kernel,
        out_type=jax.ShapeDtypeStruct((NC, npad, width), jnp.float32),
        mesh=_vector_mesh(),
        scratch_types=[
            pltpu.VMEM((n_chunks, K), jnp.int32),
            pltpu.VMEM((K, width), jnp.float32),
            pltpu.VMEM_SHARED((npad, width), jnp.float32),
            pltpu.SemaphoreType.DMA,
            pltpu.SemaphoreType.DMA,
            pltpu.SemaphoreType.DMA,
            pltpu.SemaphoreType.DMA,
        ],
    )
    def k(dst_hbm, out_hbm, idx_v, ones_v, cnt_sh, s0, s1, s2, s3):
        sems = (s0, s1, s2, s3)
        cid = lax.axis_index("c")
        sid = lax.axis_index("s")
        row_base = (cid * NS + sid) * n_chunks
        pltpu.sync_copy(dst_hbm.at[pl.ds(row_base, n_chunks)], idx_v)

        _fill(ones_v, K, width, 0.0)
        n_full = stripe // K
        tail = stripe - n_full * K

        @pl.loop(0, n_full)
        def _(t):
            pltpu.sync_copy(ones_v, cnt_sh.at[pl.ds(sid * stripe + t * K, K)])
        if tail:
            pltpu.sync_copy(ones_v.at[pl.ds(0, tail)],
                            cnt_sh.at[pl.ds(sid * stripe + n_full * K, tail)])

        _fill(ones_v, K, width, 1.0)

        plsc.subcore_barrier()

        @pl.loop(0, n_chunks, step=4)
        def _(ci):
            for j in range(4):
                @pl.when(ci + j - 4 >= 0)
                def _():
                    pltpu.make_async_copy(
                        ones_v, cnt_sh.at[idx_v.at[ci + j - 4]],
                        sems[j]).wait()
                pltpu.async_copy(ones_v, cnt_sh.at[idx_v.at[ci + j]],
                                 sems[j], add=True)
        for j in range(4):
            pltpu.make_async_copy(
                ones_v, cnt_sh.at[idx_v.at[n_chunks - 4 + j]], sems[j]).wait()

        plsc.subcore_barrier()

        @pl.loop(0, n_full)
        def _(t):
            r = sid * stripe + t * K
            pltpu.sync_copy(cnt_sh.at[pl.ds(r, K)],
                            out_hbm.at[cid].at[pl.ds(r, K)])
        if tail:
            r = sid * stripe + n_full * K
            pltpu.sync_copy(cnt_sh.at[pl.ds(r, tail)],
                            out_hbm.at[cid].at[pl.ds(r, tail)])

    return k(dst2d)


def _deg2_call(dst2d, ident, nh):
    n_rows = dst2d.shape[0]
    n_chunks = n_rows // (NC * NS)
    half = nh // 2
    hrows = nh * LANES // 128
    hhalf = hrows // 2
    stripe = hrows // NS

    cp = pltpu.CompilerParams()
    if "needs_layout_passes" in pltpu.CompilerParams.__dataclass_fields__:
        cp = dataclasses.replace(cp, needs_layout_passes=False)

    @functools.partial(
        pl.kernel,
        out_type=jax.ShapeDtypeStruct((NC, hrows, 128), jnp.float32),
        mesh=_vector_mesh(),
        compiler_params=cp,
        scratch_types=[
            pltpu.VMEM((n_chunks, K), jnp.int32),
            pltpu.VMEM((hhalf, 128), jnp.float32),
            pltpu.VMEM((hrows // 128, 128), jnp.int32),
            pltpu.VMEM_SHARED((hrows, 128), jnp.float32),
        ],
    )
    def k(dst_hbm, id_hbm, out_hbm, idx_v, hist_v, id_v, h_sh):
        cid = lax.axis_index("c")
        sid = lax.axis_index("s")
        row_base = (cid * NS + sid) * n_chunks
        pltpu.sync_copy(dst_hbm.at[pl.ds(row_base, n_chunks)], idx_v)
        pltpu.sync_copy(id_hbm, id_v)
        _fill(hist_v, hhalf, 128, 0.0)
        pltpu.sync_copy(hist_v.at[pl.ds(0, stripe)],
                        h_sh.at[pl.ds(sid * stripe, stripe)])
        plsc.subcore_barrier()

        lane = lax.broadcasted_iota(jnp.int32, (LANES,), 0)
        ones16 = jnp.ones((LANES,), jnp.float32)

        for p in range(2):
            lo = p * half

            @pl.loop(0, n_chunks)
            def _(ci):
                @pl.loop(0, K, step=LANES)
                def _(j):
                    idx = idx_v.at[ci].at[pl.ds(j, LANES)][...]
                    mask = (idx >= lo) if p else (idx < half)
                    flat = ((idx - lo) << 4) | lane
                    row = lax.shift_right_logical(flat, 7)
                    col = flat & 127
                    plsc.addupdate_scatter(hist_v, [row, col], ones16,
                                           mask=mask)

            @pl.loop(0, hhalf // 128)
            def _(t):
                pltpu.sync_copy(hist_v.at[pl.ds(t * 128, 128)],
                                h_sh.at[id_v.at[p * (hhalf // 128) + t]],
                                add=True)
            if p == 0:
                _fill(hist_v, hhalf, 128, 0.0)

        plsc.subcore_barrier()

        r = sid * stripe
        pltpu.sync_copy(h_sh.at[pl.ds(r, stripe)],
                        out_hbm.at[cid].at[pl.ds(r, stripe)])

    out = k(dst2d, ident)
    return out.reshape(NC, nh, LANES)


def _agg_call(p, ei2, npad, width):
    n_rows = ei2.shape[0]
    n_chunks = n_rows // (NC * NS)
    assert n_chunks % 8 == 0
    stripe = npad // NS

    @functools.partial(
        pl.kernel,
        out_type=jax.ShapeDtypeStruct((NC, npad, width), jnp.float32),
        mesh=_vector_mesh(),
        scratch_types=(
            [pltpu.VMEM((2, K), jnp.int32)] * 8
            + [pltpu.VMEM((K, width), jnp.float32)] * 4
            + [pltpu.VMEM_SHARED((npad, width), jnp.float32)]
            + [pltpu.SemaphoreType.DMA] * 16
        ),
    )
    def k(p_hbm, ei_hbm, out_hbm, *refs):
        idx = refs[0:8]
        rows = refs[8:12]
        s_sh = refs[12]
        g = refs[13:17]
        s = refs[17:21]
        si = refs[21:29]
        cid = lax.axis_index("c")
        sid = lax.axis_index("s")
        base = (cid * NS + sid) * n_chunks

        _fill(rows[0], K, width, 0.0)
        n_full = stripe // K
        tail = stripe - n_full * K

        @pl.loop(0, n_full)
        def _(t):
            pltpu.sync_copy(rows[0], s_sh.at[pl.ds(sid * stripe + t * K, K)])
        if tail:
            pltpu.sync_copy(rows[0].at[pl.ds(0, tail)],
                            s_sh.at[pl.ds(sid * stripe + n_full * K, tail)])

        plsc.subcore_barrier()

        for q in range(6):
            pltpu.async_copy(ei_hbm.at[base + q], idx[q], si[q])
        for b in range(2):
            pltpu.make_async_copy(ei_hbm.at[base + b], idx[b], si[b]).wait()
            pltpu.async_copy(p_hbm.at[idx[b].at[0]], rows[b], g[b])

        @pl.loop(0, n_chunks, step=8)
        def _(c):
            for u in range(8):
                b = u % 4
                br = (u + 2) % 4
                qr = (u + 2) % 8
                qp = (u + 6) % 8

                pltpu.make_async_copy(p_hbm.at[idx[u].at[0]], rows[b],
                                      g[b]).wait()
                pltpu.async_copy(rows[b], s_sh.at[idx[u].at[1]], s[b],
                                 add=True)

                @pl.when(c + u + 2 < n_chunks)
                def _():
                    @pl.when(c + u - 2 >= 0)
                    def _():
                        pltpu.make_async_copy(rows[br],
                                              s_sh.at[idx[qr].at[1]],
                                              s[br]).wait()
                    pltpu.make_async_copy(ei_hbm.at[base + c + u + 2],
                                          idx[qr], si[qr]).wait()
                    pltpu.async_copy(p_hbm.at[idx[qr].at[0]], rows[br], g[br])

                @pl.when(c + u + 6 < n_chunks)
                def _():
                    pltpu.async_copy(ei_hbm.at[base + c + u + 6], idx[qp],
                                     si[qp])

        for b in range(4):
            pltpu.make_async_copy(rows[b], s_sh.at[idx[4 + b].at[1]],
                                  s[b]).wait()

        plsc.subcore_barrier()

        @pl.loop(0, n_full)
        def _(t):
            r = sid * stripe + t * K
            pltpu.sync_copy(s_sh.at[pl.ds(r, K)],
                            out_hbm.at[cid].at[pl.ds(r, K)])
        if tail:
            r = sid * stripe + n_full * K
            pltpu.sync_copy(s_sh.at[pl.ds(r, tail)],
                            out_hbm.at[cid].at[pl.ds(r, tail)])

    return k(p, ei2)


def _dinv_block(c0, c1):
    deg = c0[0].sum(axis=-1) + c1[0].sum(axis=-1) + 1.0
    return lax.rsqrt(deg)


def _p0_call(x, w0, cnt, npad):
    n, d = x.shape
    h = w0.shape[1]

    def body(x_ref, w_ref, c0_ref, c1_ref, p_ref):
        dinv = _dinv_block(c0_ref, c1_ref)
        hw = jnp.dot(x_ref[...], w_ref[...], preferred_element_type=jnp.float32)
        p_ref[...] = hw * dinv[:, None]

    return pl.pallas_call(
        body,
        grid=(n // TCB,),
        in_specs=[
            pl.BlockSpec((TCB, d), lambda i: (i, 0)),
            pl.BlockSpec((d, h), lambda i: (0, 0)),
            pl.BlockSpec((1, TCB, 16), lambda i: (0, i, 0)),
            pl.BlockSpec((1, TCB, 16), lambda i: (1, i, 0)),
        ],
        out_specs=pl.BlockSpec((TCB, h), lambda i: (i, 0)),
        out_shape=jax.ShapeDtypeStruct((n, h), jnp.float32),
    )(x, w0, cnt, cnt)


def _p1_call(s0, p0, cnt, b0, npad):
    n, h = p0.shape

    def body(s0a, s0b, p0_ref, c0_ref, c1_ref, b_ref, p1_ref):
        dinv = _dinv_block(c0_ref, c1_ref)
        hmat = (s0a[0] + s0b[0] + p0_ref[...]) * dinv[:, None] + b_ref[...]
        hmat = jnp.maximum(hmat, 0.0)
        p1_ref[...] = hmat * dinv[:, None]

    return pl.pallas_call(
        body,
        grid=(n // TCB,),
        in_specs=[
            pl.BlockSpec((1, TCB, h), lambda i: (0, i, 0)),
            pl.BlockSpec((1, TCB, h), lambda i: (1, i, 0)),
            pl.BlockSpec((TCB, h), lambda i: (i, 0)),
            pl.BlockSpec((1, TCB, 16), lambda i: (0, i, 0)),
            pl.BlockSpec((1, TCB, 16), lambda i: (1, i, 0)),
            pl.BlockSpec((1, h), lambda i: (0, 0)),
        ],
        out_specs=pl.BlockSpec((TCB, h), lambda i: (i, 0)),
        out_shape=jax.ShapeDtypeStruct((n, h), jnp.float32),
    )(s0, s0, p0, cnt, cnt, b0)


def _out_call(s1, p1, cnt, w1, b1, npad):
    n, h = p1.shape
    c = w1.shape[1]

    def body(s1a, s1b, p1_ref, c0_ref, c1_ref, w_ref, b_ref, o_ref):
        dinv = _dinv_block(c0_ref, c1_ref)
        agg = (s1a[0] + s1b[0] + p1_ref[...]) * dinv[:, None]
        o_ref[...] = jnp.dot(agg, w_ref[...],
                             preferred_element_type=jnp.float32) + b_ref[...]

    return pl.pallas_call(
        body,
        grid=(n // TCB,),
        in_specs=[
            pl.BlockSpec((1, TCB, h), lambda i: (0, i, 0)),
            pl.BlockSpec((1, TCB, h), lambda i: (1, i, 0)),
            pl.BlockSpec((TCB, h), lambda i: (i, 0)),
            pl.BlockSpec((1, TCB, 16), lambda i: (0, i, 0)),
            pl.BlockSpec((1, TCB, 16), lambda i: (1, i, 0)),
            pl.BlockSpec((h, c), lambda i: (0, 0)),
            pl.BlockSpec((1, c), lambda i: (0, 0)),
        ],
        out_specs=pl.BlockSpec((TCB, c), lambda i: (i, 0)),
        out_shape=jax.ShapeDtypeStruct((n, c), jnp.float32),
    )(s1, s1, p1, cnt, cnt, w1, b1)


def kernel(x, edge_index, W0, b0, W1, b1):
    n, d = x.shape
    h = W0.shape[1]

    src, dst = edge_index[0], edge_index[1]
    e = src.shape[0]
    chunk_total = NC * NS * K * 8
    epad = ((e + chunk_total - 1) // chunk_total) * chunk_total
    npad = ((n + 1 + 127) // 128) * 128

    pad = epad - e
    arp = jnp.arange(pad, dtype=src.dtype)
    src_p = jnp.concatenate([src, n + arp % K]).reshape(-1, K)
    dst_p = jnp.concatenate([dst, arp % n]).reshape(-1, K)
    dst_deg = jnp.concatenate([dst, n + arp % (npad - n)]).reshape(-1, K)
    ei2 = jnp.stack([src_p, dst_p], axis=1)
    b0r = b0.reshape(1, h)
    b1r = b1.reshape(1, b1.shape[0])
    zrows = jnp.zeros((K, h), jnp.float32)

    nh = 10240
    ident = jnp.arange(nh * LANES // 128, dtype=jnp.int32).reshape(-1, 128)
    cnt = _deg2_call(dst_deg, ident, nh)
    p0 = _p0_call(x, W0, cnt, npad)
    s0 = _agg_call(jnp.concatenate([p0, zrows]), ei2, npad, h)
    p1 = _p1_call(s0, p0, cnt, b0r, npad)
    s1 = _agg_call(jnp.concatenate([p1, zrows]), ei2, npad, h)
    return _out_call(s1, p1, cnt, W1, b1r, npad)


# speedup vs baseline: 1.1789x; 1.0479x over previous
import dataclasses
import functools

import jax
import jax.numpy as jnp
from jax import lax
from jax.experimental import pallas as pl
from jax.experimental.pallas import tpu as pltpu
from jax.experimental.pallas import tpu_sc as plsc

NC = 2
NS = 16
LANES = 16
K = 80
TCB = 400


def _vector_mesh():
    return plsc.VectorSubcoreMesh(core_axis_name="c", subcore_axis_name="s")


def _fill(ref, rows, width, value):
    @pl.loop(0, rows)
    def _(i):
        @pl.loop(0, width, step=LANES)
        def _(j):
            ref.at[i].at[pl.ds(j, LANES)][...] = jnp.full((LANES,), value,
                                                          jnp.float32)


def _deg_call(dst2d, npad, width):
    n_rows = dst2d.shape[0]
    n_chunks = n_rows // (NC * NS)
    stripe = npad // NS

    @functools.partial(
        pl.kernel,
        out_type=jax.ShapeDtypeStruct((NC, npad, width), jnp.float32),
        mesh=_vector_mesh(),
        scratch_types=[
            pltpu.VMEM((n_chunks, K), jnp.int32),
            pltpu.VMEM((K, width), jnp.float32),
            pltpu.VMEM_SHARED((npad, width), jnp.float32),
            pltpu.SemaphoreType.DMA,
            pltpu.SemaphoreType.DMA,
            pltpu.SemaphoreType.DMA,
            pltpu.SemaphoreType.DMA,
        ],
    )
    def k(dst_hbm, out_hbm, idx_v, ones_v, cnt_sh, s0, s1, s2, s3):
        sems = (s0, s1, s2, s3)
        cid = lax.axis_index("c")
        sid = lax.axis_index("s")
        row_base = (cid * NS + sid) * n_chunks
        pltpu.sync_copy(dst_hbm.at[pl.ds(row_base, n_chunks)], idx_v)

        _fill(ones_v, K, width, 0.0)
        n_full = stripe // K
        tail = stripe - n_full * K

        @pl.loop(0, n_full)
        def _(t):
            pltpu.sync_copy(ones_v, cnt_sh.at[pl.ds(sid * stripe + t * K, K)])
        if tail:
            pltpu.sync_copy(ones_v.at[pl.ds(0, tail)],
                            cnt_sh.at[pl.ds(sid * stripe + n_full * K, tail)])

        _fill(ones_v, K, width, 1.0)

        plsc.subcore_barrier()

        @pl.loop(0, n_chunks, step=4)
        def _(ci):
            for j in range(4):
                @pl.when(ci + j - 4 >= 0)
                def _():
                    pltpu.make_async_copy(
                        ones_v, cnt_sh.at[idx_v.at[ci + j - 4]],
                        sems[j]).wait()
                pltpu.async_copy(ones_v, cnt_sh.at[idx_v.at[ci + j]],
                                 sems[j], add=True)
        for j in range(4):
            pltpu.make_async_copy(
                ones_v, cnt_sh.at[idx_v.at[n_chunks - 4 + j]], sems[j]).wait()

        plsc.subcore_barrier()

        @pl.loop(0, n_full)
        def _(t):
            r = sid * stripe + t * K
            pltpu.sync_copy(cnt_sh.at[pl.ds(r, K)],
                            out_hbm.at[cid].at[pl.ds(r, K)])
        if tail:
            r = sid * stripe + n_full * K
            pltpu.sync_copy(cnt_sh.at[pl.ds(r, tail)],
                            out_hbm.at[cid].at[pl.ds(r, tail)])

    return k(dst2d)


def _deg2_call(dst2d, ident, nh):
    n_rows = dst2d.shape[0]
    n_chunks = n_rows // (NC * NS)
    half = nh // 2
    hrows = nh * LANES // 128
    hhalf = hrows // 2
    stripe = hrows // NS

    cp = pltpu.CompilerParams()
    if "needs_layout_passes" in pltpu.CompilerParams.__dataclass_fields__:
        cp = dataclasses.replace(cp, needs_layout_passes=False)

    @functools.partial(
        pl.kernel,
        out_type=jax.ShapeDtypeStruct((NC, hrows, 128), jnp.float32),
        mesh=_vector_mesh(),
        compiler_params=cp,
        scratch_types=[
            pltpu.VMEM((n_chunks, K), jnp.int32),
            pltpu.VMEM((hhalf, 128), jnp.float32),
            pltpu.VMEM((hrows // 128, 128), jnp.int32),
            pltpu.VMEM_SHARED((hrows, 128), jnp.float32),
        ],
    )
    def k(dst_hbm, id_hbm, out_hbm, idx_v, hist_v, id_v, h_sh):
        cid = lax.axis_index("c")
        sid = lax.axis_index("s")
        row_base = (cid * NS + sid) * n_chunks
        pltpu.sync_copy(dst_hbm.at[pl.ds(row_base, n_chunks)], idx_v)
        pltpu.sync_copy(id_hbm, id_v)
        _fill(hist_v, hhalf, 128, 0.0)
        pltpu.sync_copy(hist_v.at[pl.ds(0, stripe)],
                        h_sh.at[pl.ds(sid * stripe, stripe)])
        plsc.subcore_barrier()

        lane = lax.broadcasted_iota(jnp.int32, (LANES,), 0)
        ones16 = jnp.ones((LANES,), jnp.float32)

        for p in range(2):
            lo = p * half

            @pl.loop(0, n_chunks)
            def _(ci):
                @pl.loop(0, K, step=LANES)
                def _(j):
                    idx = idx_v.at[ci].at[pl.ds(j, LANES)][...]
                    mask = (idx >= lo) if p else (idx < half)
                    flat = ((idx - lo) << 4) | lane
                    row = lax.shift_right_logical(flat, 7)
                    col = flat & 127
                    plsc.addupdate_scatter(hist_v, [row, col], ones16,
                                           mask=mask)

            @pl.loop(0, hhalf // 128)
            def _(t):
                pltpu.sync_copy(hist_v.at[pl.ds(t * 128, 128)],
                                h_sh.at[id_v.at[p * (hhalf // 128) + t]],
                                add=True)
            if p == 0:
                _fill(hist_v, hhalf, 128, 0.0)

        plsc.subcore_barrier()

        r = sid * stripe
        pltpu.sync_copy(h_sh.at[pl.ds(r, stripe)],
                        out_hbm.at[cid].at[pl.ds(r, stripe)])

    out = k(dst2d, ident)
    return out.reshape(NC, nh, LANES)


def _agg_call(p, ei2, npad, width):
    n_rows = ei2.shape[0]
    n_chunks = n_rows // (NC * NS)
    assert n_chunks % 8 == 0
    stripe = npad // NS

    @functools.partial(
        pl.kernel,
        out_type=jax.ShapeDtypeStruct((NC, npad, width), jnp.float32),
        mesh=_vector_mesh(),
        scratch_types=(
            [pltpu.VMEM((2, K), jnp.int32)] * 8
            + [pltpu.VMEM((K, width), jnp.float32)] * 4
            + [pltpu.VMEM_SHARED((npad, width), jnp.float32)]
            + [pltpu.SemaphoreType.DMA] * 16
        ),
    )
    def k(p_hbm, ei_hbm, out_hbm, *refs):
        idx = refs[0:8]
        rows = refs[8:12]
        s_sh = refs[12]
        g = refs[13:17]
        s = refs[17:21]
        si = refs[21:29]
        cid = lax.axis_index("c")
        sid = lax.axis_index("s")
        base = (cid * NS + sid) * n_chunks

        for q in range(6):
            pltpu.async_copy(ei_hbm.at[base + q], idx[q], si[q])
        for b in range(2):
            pltpu.make_async_copy(ei_hbm.at[base + b], idx[b], si[b]).wait()
            pltpu.async_copy(p_hbm.at[idx[b].at[0]], rows[b], g[b])

        _fill(rows[3], K, width, 0.0)
        n_full = stripe // K
        tail = stripe - n_full * K

        @pl.loop(0, n_full)
        def _(t):
            pltpu.sync_copy(rows[3], s_sh.at[pl.ds(sid * stripe + t * K, K)])
        if tail:
            pltpu.sync_copy(rows[3].at[pl.ds(0, tail)],
                            s_sh.at[pl.ds(sid * stripe + n_full * K, tail)])

        plsc.subcore_barrier()

        @pl.loop(0, n_chunks, step=8)
        def _(c):
            for u in range(8):
                b = u % 4
                br = (u + 2) % 4
                qr = (u + 2) % 8
                qp = (u + 6) % 8

                pltpu.make_async_copy(p_hbm.at[idx[u].at[0]], rows[b],
                                      g[b]).wait()
                pltpu.async_copy(rows[b], s_sh.at[idx[u].at[1]], s[b],
                                 add=True)

                @pl.when(c + u + 2 < n_chunks)
                def _():
                    @pl.when(c + u - 2 >= 0)
                    def _():
                        pltpu.make_async_copy(rows[br],
                                              s_sh.at[idx[qr].at[1]],
                                              s[br]).wait()
                    pltpu.make_async_copy(ei_hbm.at[base + c + u + 2],
                                          idx[qr], si[qr]).wait()
                    pltpu.async_copy(p_hbm.at[idx[qr].at[0]], rows[br], g[br])

                @pl.when(c + u + 6 < n_chunks)
                def _():
                    pltpu.async_copy(ei_hbm.at[base + c + u + 6], idx[qp],
                                     si[qp])

        for b in range(4):
            pltpu.make_async_copy(rows[b], s_sh.at[idx[4 + b].at[1]],
                                  s[b]).wait()

        plsc.subcore_barrier()

        @pl.loop(0, n_full)
        def _(t):
            r = sid * stripe + t * K
            pltpu.sync_copy(s_sh.at[pl.ds(r, K)],
                            out_hbm.at[cid].at[pl.ds(r, K)])
        if tail:
            r = sid * stripe + n_full * K
            pltpu.sync_copy(s_sh.at[pl.ds(r, tail)],
                            out_hbm.at[cid].at[pl.ds(r, tail)])

    return k(p, ei2)


def _dinv_block(c0, c1):
    deg = c0[0].sum(axis=-1) + c1[0].sum(axis=-1) + 1.0
    return lax.rsqrt(deg)


def _p0_call(x, w0, cnt, npad):
    n, d = x.shape
    h = w0.shape[1]

    def body(x_ref, w_ref, c0_ref, c1_ref, p_ref):
        dinv = _dinv_block(c0_ref, c1_ref)
        hw = jnp.dot(x_ref[...], w_ref[...], preferred_element_type=jnp.float32)
        p_ref[...] = hw * dinv[:, None]

    return pl.pallas_call(
        body,
        grid=(n // TCB,),
        in_specs=[
            pl.BlockSpec((TCB, d), lambda i: (i, 0)),
            pl.BlockSpec((d, h), lambda i: (0, 0)),
            pl.BlockSpec((1, TCB, 16), lambda i: (0, i, 0)),
            pl.BlockSpec((1, TCB, 16), lambda i: (1, i, 0)),
        ],
        out_specs=pl.BlockSpec((TCB, h), lambda i: (i, 0)),
        out_shape=jax.ShapeDtypeStruct((n, h), jnp.float32),
    )(x, w0, cnt, cnt)


def _p1_call(s0, p0, cnt, b0, npad):
    n, h = p0.shape

    def body(s0a, s0b, p0_ref, c0_ref, c1_ref, b_ref, p1_ref):
        dinv = _dinv_block(c0_ref, c1_ref)
        hmat = (s0a[0] + s0b[0] + p0_ref[...]) * dinv[:, None] + b_ref[...]
        hmat = jnp.maximum(hmat, 0.0)
        p1_ref[...] = hmat * dinv[:, None]

    return pl.pallas_call(
        body,
        grid=(n // TCB,),
        in_specs=[
            pl.BlockSpec((1, TCB, h), lambda i: (0, i, 0)),
            pl.BlockSpec((1, TCB, h), lambda i: (1, i, 0)),
            pl.BlockSpec((TCB, h), lambda i: (i, 0)),
            pl.BlockSpec((1, TCB, 16), lambda i: (0, i, 0)),
            pl.BlockSpec((1, TCB, 16), lambda i: (1, i, 0)),
            pl.BlockSpec((1, h), lambda i: (0, 0)),
        ],
        out_specs=pl.BlockSpec((TCB, h), lambda i: (i, 0)),
        out_shape=jax.ShapeDtypeStruct((n, h), jnp.float32),
    )(s0, s0, p0, cnt, cnt, b0)


def _out_call(s1, p1, cnt, w1, b1, npad):
    n, h = p1.shape
    c = w1.shape[1]

    def body(s1a, s1b, p1_ref, c0_ref, c1_ref, w_ref, b_ref, o_ref):
        dinv = _dinv_block(c0_ref, c1_ref)
        agg = (s1a[0] + s1b[0] + p1_ref[...]) * dinv[:, None]
        o_ref[...] = jnp.dot(agg, w_ref[...],
                             preferred_element_type=jnp.float32) + b_ref[...]

    return pl.pallas_call(
        body,
        grid=(n // TCB,),
        in_specs=[
            pl.BlockSpec((1, TCB, h), lambda i: (0, i, 0)),
            pl.BlockSpec((1, TCB, h), lambda i: (1, i, 0)),
            pl.BlockSpec((TCB, h), lambda i: (i, 0)),
            pl.BlockSpec((1, TCB, 16), lambda i: (0, i, 0)),
            pl.BlockSpec((1, TCB, 16), lambda i: (1, i, 0)),
            pl.BlockSpec((h, c), lambda i: (0, 0)),
            pl.BlockSpec((1, c), lambda i: (0, 0)),
        ],
        out_specs=pl.BlockSpec((TCB, c), lambda i: (i, 0)),
        out_shape=jax.ShapeDtypeStruct((n, c), jnp.float32),
    )(s1, s1, p1, cnt, cnt, w1, b1)


def kernel(x, edge_index, W0, b0, W1, b1):
    n, d = x.shape
    h = W0.shape[1]

    src, dst = edge_index[0], edge_index[1]
    e = src.shape[0]
    chunk_total = NC * NS * K * 8
    epad = ((e + chunk_total - 1) // chunk_total) * chunk_total
    npad = ((n + 1 + 127) // 128) * 128

    pad = epad - e
    arp = jnp.arange(pad, dtype=src.dtype)
    src_p = jnp.concatenate([src, n + arp % K]).reshape(-1, K)
    dst_p = jnp.concatenate([dst, arp % n]).reshape(-1, K)
    dst_deg = jnp.concatenate([dst, n + arp % (npad - n)]).reshape(-1, K)
    ei2 = jnp.stack([src_p, dst_p], axis=1)
    b0r = b0.reshape(1, h)
    b1r = b1.reshape(1, b1.shape[0])
    zrows = jnp.zeros((K, h), jnp.float32)

    nh = 10240
    ident = jnp.arange(nh * LANES // 128, dtype=jnp.int32).reshape(-1, 128)
    cnt = _deg2_call(dst_deg, ident, nh)
    p0 = _p0_call(x, W0, cnt, npad)
    s0 = _agg_call(jnp.concatenate([p0, zrows]), ei2, npad, h)
    p1 = _p1_call(s0, p0, cnt, b0r, npad)
    s1 = _agg_call(jnp.concatenate([p1, zrows]), ei2, npad, h)
    return _out_call(s1, p1, cnt, W1, b1r, npad)
